# baseline (speedup 1.0000x reference)
import functools

import jax
import jax.numpy as jnp
from jax import lax
from jax.experimental import pallas as pl
from jax.experimental.pallas import tpu as pltpu
from jax.experimental.pallas import tpu_sc as plsc

N = 10000
E = 320000
D_IN = 128
H = 64
OUT = 10
G = 64

CH = 128
NTILES = 32
EPT = E // NTILES
NC_T = 80
NBUF = 8
SPLIT = 80
N_PAD = 10240
ROWS_PER_TILE = N_PAD // 16


def _elu(v):
    return jnp.where(v > 0, v, jnp.exp(jnp.minimum(v, 0.0)) - 1.0)


def _sc_segsum(y, src2, dst2, zrows):
    mesh = plsc.VectorSubcoreMesh(core_axis_name="c", subcore_axis_name="s")

    @functools.partial(
        pl.kernel,
        out_type=jax.ShapeDtypeStruct((2, N_PAD, H), jnp.bfloat16),
        mesh=mesh,
        scratch_types=[
            pltpu.VMEM((NC_T, CH), jnp.int32),
            pltpu.VMEM((NC_T, CH), jnp.int32),
            pltpu.VMEM_SHARED((N_PAD, H), jnp.bfloat16),
            pltpu.VMEM_SHARED((N_PAD, H), jnp.bfloat16),
        ] + [pltpu.VMEM((CH, H), jnp.bfloat16) for _ in range(NBUF)]
        + [pltpu.SemaphoreType.DMA for _ in range(2 * NBUF)],
        compiler_params=pltpu.CompilerParams(use_tc_tiling_on_sc=False),
    )
    def k(y_hbm, src_hbm, dst_hbm, z_hbm, out_hbm, src_all, dst_all, acc_sh,
          ytab_sh, *rest):
        rows = rest[:NBUF]
        gsem = rest[NBUF:2 * NBUF]
        ssem = rest[2 * NBUF:]
        cid = lax.axis_index("c")
        sid = lax.axis_index("s")
        wid = sid * 2 + cid
        row0 = sid * ROWS_PER_TILE
        pltpu.async_copy(src_hbm.at[wid], src_all, gsem[0])
        pltpu.async_copy(dst_hbm.at[wid], dst_all, gsem[1])
        pltpu.async_copy(z_hbm, acc_sh.at[pl.ds(row0, ROWS_PER_TILE)],
                         gsem[2])
        pltpu.async_copy(y_hbm.at[pl.ds(row0, ROWS_PER_TILE)],
                         ytab_sh.at[pl.ds(row0, ROWS_PER_TILE)], gsem[3])
        pltpu.make_async_copy(src_hbm.at[wid], src_all, gsem[0]).wait()
        pltpu.make_async_copy(dst_hbm.at[wid], dst_all, gsem[1]).wait()
        pltpu.make_async_copy(z_hbm, acc_sh.at[pl.ds(row0, ROWS_PER_TILE)],
                              gsem[2]).wait()
        pltpu.make_async_copy(y_hbm.at[pl.ds(row0, ROWS_PER_TILE)],
                              ytab_sh.at[pl.ds(row0, ROWS_PER_TILE)],
                              gsem[3]).wait()
        plsc.subcore_barrier()

        @pl.loop(0, NC_T // NBUF)
        def _(k_):
            base = k_ * NBUF
            for b in range(NBUF):
                j = base + b

                @pl.when(k_ > 0)
                def _():
                    pltpu.make_async_copy(
                        rows[b], acc_sh.at[dst_all.at[j - NBUF]],
                        ssem[b]).wait()

                @pl.when(j < SPLIT)
                def _():
                    pltpu.async_copy(y_hbm.at[src_all.at[j]], rows[b],
                                     gsem[b])

                @pl.when(j >= SPLIT)
                def _():
                    pltpu.async_copy(ytab_sh.at[src_all.at[j]], rows[b],
                                     gsem[b])
            for b in range(NBUF):
                j = base + b
                pltpu.make_async_copy(
                    ytab_sh.at[src_all.at[j]], rows[b], gsem[b]).wait()
                pltpu.async_copy(rows[b], acc_sh.at[dst_all.at[j]], ssem[b],
                                 add=True)

        for b in range(NBUF):
            pltpu.make_async_copy(
                rows[b], acc_sh.at[dst_all.at[NC_T - NBUF + b]],
                ssem[b]).wait()
        plsc.subcore_barrier()
        pltpu.sync_copy(acc_sh.at[pl.ds(row0, ROWS_PER_TILE)],
                        out_hbm.at[cid, pl.ds(row0, ROWS_PER_TILE)])

    return k(y, src2, dst2, zrows)


def _tc(body, out_shape, *args):
    return pl.pallas_call(body, out_shape=out_shape)(*args)


def _f32(shape):
    return jax.ShapeDtypeStruct(shape, jnp.float32)


def _dual(shape):
    return [jax.ShapeDtypeStruct(shape, jnp.float32),
            jax.ShapeDtypeStruct((N_PAD, shape[1]), jnp.bfloat16)]


def _proj_body(x_ref, w_ref, o_ref, ob_ref):
    y = jnp.dot(x_ref[...], w_ref[...], preferred_element_type=jnp.float32)
    o_ref[...] = y
    ob_ref[:N] = y.astype(jnp.bfloat16)
    ob_ref[N:] = jnp.zeros((N_PAD - N, H), jnp.bfloat16)


def _pre_body(y_ref, p_ref, b1_ref, g_ref, bb_ref, w2_ref, b2_ref, o_ref,
              ob_ref):
    h = (y_ref[...] + p_ref[0, :N].astype(jnp.float32)
         + p_ref[1, :N].astype(jnp.float32) + b1_ref[...])
    mu = jnp.mean(h, axis=0, keepdims=True)
    var = jnp.mean((h - mu) * (h - mu), axis=0, keepdims=True)
    h = (h - mu) * lax.rsqrt(var + 1e-5) * g_ref[...] + bb_ref[...]
    h = _elu(h)
    h = jnp.dot(h, w2_ref[...], preferred_element_type=jnp.float32) + b2_ref[...]
    h = _elu(h)
    o_ref[...] = h
    ob_ref[:N] = h.astype(jnp.bfloat16)
    ob_ref[N:] = jnp.zeros((N_PAD - N, H), jnp.bfloat16)


def _mid_body(h_ref, p_ref, w1_ref, b1_ref, w2_ref, b2_ref, o_ref, ob_ref):
    h = (h_ref[...] + p_ref[0, :N].astype(jnp.float32)
         + p_ref[1, :N].astype(jnp.float32))
    h = _elu(jnp.dot(h, w1_ref[...], preferred_element_type=jnp.float32)
             + b1_ref[...])
    h = jnp.dot(h, w2_ref[...], preferred_element_type=jnp.float32) + b2_ref[...]
    h = _elu(h)
    o_ref[...] = h
    ob_ref[:N] = h.astype(jnp.bfloat16)
    ob_ref[N:] = jnp.zeros((N_PAD - N, H), jnp.bfloat16)


def _post_body(h_ref, p_ref, batch_ref, w1_ref, b1_ref, w2_ref, b2_ref,
               fw1_ref, fb1_ref, fw2_ref, fb2_ref, fw3_ref, fb3_ref, o_ref):
    h = (h_ref[...] + p_ref[0, :N].astype(jnp.float32)
         + p_ref[1, :N].astype(jnp.float32))
    h = _elu(jnp.dot(h, w1_ref[...], preferred_element_type=jnp.float32)
             + b1_ref[...])
    h = jnp.dot(h, w2_ref[...], preferred_element_type=jnp.float32) + b2_ref[...]
    h = _elu(h)
    gids = lax.broadcasted_iota(jnp.int32, (G, N), 0)
    mask = (gids == batch_ref[...]).astype(jnp.float32)
    sums = jnp.dot(mask, h, preferred_element_type=jnp.float32)
    cnt = jnp.sum(mask, axis=1, keepdims=True)
    pooled = sums / jnp.maximum(cnt, 1.0)
    z = _elu(jnp.dot(pooled, fw1_ref[...], preferred_element_type=jnp.float32)
             + fb1_ref[...])
    z = _elu(jnp.dot(z, fw2_ref[...], preferred_element_type=jnp.float32)
             + fb2_ref[...])
    o_ref[...] = jnp.dot(z, fw3_ref[...],
                         preferred_element_type=jnp.float32) + fb3_ref[...]


def kernel(x, edge_index, batch, pre_W1, pre_b1, bn_g, bn_b, pre_W2, pre_b2,
           mid_W1, mid_b1, mid_W2, mid_b2, post_W1, post_b1, post_W2, post_b2,
           f_W1, f_b1, f_W2, f_b2, f_W3, f_b3):
    pad = NC_T * CH - EPT
    src2 = jnp.pad(edge_index[0].reshape(NTILES, EPT),
                   ((0, 0), (0, pad))).reshape(NTILES, NC_T, CH)
    dst2 = jnp.pad(edge_index[1].reshape(NTILES, EPT),
                   ((0, 0), (0, pad)),
                   constant_values=N).reshape(NTILES, NC_T, CH)
    zrows = jnp.zeros((ROWS_PER_TILE, H), jnp.bfloat16)
    batch_row = batch.reshape(1, N)
    r1 = lambda v: v.reshape(1, -1)

    y, yb = _tc(_proj_body, _dual((N, H)), x, pre_W1)
    p = _sc_segsum(yb, src2, dst2, zrows)
    h1, h1b = _tc(_pre_body, _dual((N, H)), y, p, r1(pre_b1), r1(bn_g),
                  r1(bn_b), pre_W2, r1(pre_b2))
    q = _sc_segsum(h1b, src2, dst2, zrows)
    h2, h2b = _tc(_mid_body, _dual((N, H)), h1, q, mid_W1, r1(mid_b1),
                  mid_W2, r1(mid_b2))
    r = _sc_segsum(h2b, src2, dst2, zrows)
    out = _tc(_post_body, _f32((G, OUT)), h2, r, batch_row,
              post_W1, r1(post_b1), post_W2, r1(post_b2),
              f_W1, r1(f_b1), f_W2, r1(f_b2), f_W3, r1(f_b3))
    return out


# speedup vs baseline: 1.8664x; 1.8664x over previous
import functools

import jax
import jax.numpy as jnp
from jax import lax
from jax.experimental import pallas as pl
from jax.experimental.pallas import tpu as pltpu
from jax.experimental.pallas import tpu_sc as plsc

N = 10000
E = 320000
D_IN = 128
H = 64
OUT = 10
G = 64

CH = 128
NTILES = 32
EPT = E // NTILES
NC_T = 80
NBUF = 8
SPLIT = 76
N_PAD = 10240
ROWS_PER_TILE = N_PAD // 16


def _elu(v):
    return jnp.where(v > 0, v, jnp.exp(jnp.minimum(v, 0.0)) - 1.0)


def _sc_segsum(y, src2, dst2, zrows):
    mesh = plsc.VectorSubcoreMesh(core_axis_name="c", subcore_axis_name="s")

    @functools.partial(
        pl.kernel,
        out_type=jax.ShapeDtypeStruct((2, N_PAD, H), jnp.bfloat16),
        mesh=mesh,
        scratch_types=[
            pltpu.VMEM((NC_T, CH), jnp.int32),
            pltpu.VMEM((NC_T, CH), jnp.int32),
            pltpu.VMEM_SHARED((N_PAD, H), jnp.bfloat16),
            pltpu.VMEM_SHARED((N_PAD, H), jnp.bfloat16),
        ] + [pltpu.VMEM((CH, H), jnp.bfloat16) for _ in range(NBUF)]
        + [pltpu.SemaphoreType.DMA for _ in range(2 * NBUF)],
        compiler_params=pltpu.CompilerParams(use_tc_tiling_on_sc=False),
    )
    def k(y_hbm, src_hbm, dst_hbm, z_hbm, out_hbm, src_all, dst_all, acc_sh,
          ytab_sh, *rest):
        rows = rest[:NBUF]
        gsem = rest[NBUF:2 * NBUF]
        ssem = rest[2 * NBUF:]
        cid = lax.axis_index("c")
        sid = lax.axis_index("s")
        wid = sid * 2 + cid
        row0 = sid * ROWS_PER_TILE
        pltpu.async_copy(src_hbm.at[wid], src_all, gsem[0])
        pltpu.async_copy(dst_hbm.at[wid], dst_all, gsem[1])
        pltpu.async_copy(z_hbm, acc_sh.at[pl.ds(row0, ROWS_PER_TILE)],
                         gsem[2])
        pltpu.async_copy(y_hbm.at[pl.ds(row0, ROWS_PER_TILE)],
                         ytab_sh.at[pl.ds(row0, ROWS_PER_TILE)], gsem[3])
        pltpu.make_async_copy(src_hbm.at[wid], src_all, gsem[0]).wait()
        pltpu.make_async_copy(dst_hbm.at[wid], dst_all, gsem[1]).wait()
        pltpu.make_async_copy(z_hbm, acc_sh.at[pl.ds(row0, ROWS_PER_TILE)],
                              gsem[2]).wait()
        pltpu.make_async_copy(y_hbm.at[pl.ds(row0, ROWS_PER_TILE)],
                              ytab_sh.at[pl.ds(row0, ROWS_PER_TILE)],
                              gsem[3]).wait()
        plsc.subcore_barrier()

        @pl.loop(0, NC_T // NBUF)
        def _(k_):
            base = k_ * NBUF
            for b in range(NBUF):
                j = base + b

                @pl.when(k_ > 0)
                def _():
                    pltpu.make_async_copy(
                        rows[b], acc_sh.at[dst_all.at[j - NBUF]],
                        ssem[b]).wait()

                @pl.when(j < SPLIT)
                def _():
                    pltpu.async_copy(y_hbm.at[src_all.at[j]], rows[b],
                                     gsem[b])

                @pl.when(j >= SPLIT)
                def _():
                    pltpu.async_copy(ytab_sh.at[src_all.at[j]], rows[b],
                                     gsem[b])
            for b in range(NBUF):
                j = base + b
                pltpu.make_async_copy(
                    ytab_sh.at[src_all.at[j]], rows[b], gsem[b]).wait()
                pltpu.async_copy(rows[b], acc_sh.at[dst_all.at[j]], ssem[b],
                                 add=True)

        for b in range(NBUF):
            pltpu.make_async_copy(
                rows[b], acc_sh.at[dst_all.at[NC_T - NBUF + b]],
                ssem[b]).wait()
        plsc.subcore_barrier()
        pltpu.sync_copy(acc_sh.at[pl.ds(row0, ROWS_PER_TILE)],
                        out_hbm.at[cid, pl.ds(row0, ROWS_PER_TILE)])

    return k(y, src2, dst2, zrows)


def _tc(body, out_shape, *args):
    return pl.pallas_call(body, out_shape=out_shape)(*args)


def _f32(shape):
    return jax.ShapeDtypeStruct(shape, jnp.float32)


def _dual(shape):
    return [jax.ShapeDtypeStruct(shape, jnp.float32),
            jax.ShapeDtypeStruct((N_PAD, shape[1]), jnp.bfloat16)]


def _proj_body(x_ref, w_ref, o_ref, ob_ref):
    y = jnp.dot(x_ref[...], w_ref[...], preferred_element_type=jnp.float32)
    o_ref[...] = y
    ob_ref[:N] = y.astype(jnp.bfloat16)
    ob_ref[N:] = jnp.zeros((N_PAD - N, H), jnp.bfloat16)


def _pre_body(y_ref, p_ref, b1_ref, g_ref, bb_ref, w2_ref, b2_ref, o_ref,
              ob_ref):
    h = (y_ref[...] + p_ref[0, :N].astype(jnp.float32)
         + p_ref[1, :N].astype(jnp.float32) + b1_ref[...])
    mu = jnp.mean(h, axis=0, keepdims=True)
    var = jnp.mean((h - mu) * (h - mu), axis=0, keepdims=True)
    h = (h - mu) * lax.rsqrt(var + 1e-5) * g_ref[...] + bb_ref[...]
    h = _elu(h)
    h = jnp.dot(h, w2_ref[...], preferred_element_type=jnp.float32) + b2_ref[...]
    h = _elu(h)
    o_ref[...] = h
    ob_ref[:N] = h.astype(jnp.bfloat16)
    ob_ref[N:] = jnp.zeros((N_PAD - N, H), jnp.bfloat16)


def _mid_body(h_ref, p_ref, w1_ref, b1_ref, w2_ref, b2_ref, o_ref, ob_ref):
    h = (h_ref[...] + p_ref[0, :N].astype(jnp.float32)
         + p_ref[1, :N].astype(jnp.float32))
    h = _elu(jnp.dot(h, w1_ref[...], preferred_element_type=jnp.float32)
             + b1_ref[...])
    h = jnp.dot(h, w2_ref[...], preferred_element_type=jnp.float32) + b2_ref[...]
    h = _elu(h)
    o_ref[...] = h
    ob_ref[:N] = h.astype(jnp.bfloat16)
    ob_ref[N:] = jnp.zeros((N_PAD - N, H), jnp.bfloat16)


def _post_body(h_ref, p_ref, batch_ref, w1_ref, b1_ref, w2_ref, b2_ref,
               fw1_ref, fb1_ref, fw2_ref, fb2_ref, fw3_ref, fb3_ref, o_ref):
    h = (h_ref[...] + p_ref[0, :N].astype(jnp.float32)
         + p_ref[1, :N].astype(jnp.float32))
    h = _elu(jnp.dot(h, w1_ref[...], preferred_element_type=jnp.float32)
             + b1_ref[...])
    h = jnp.dot(h, w2_ref[...], preferred_element_type=jnp.float32) + b2_ref[...]
    h = _elu(h)
    gids = lax.broadcasted_iota(jnp.int32, (G, N), 0)
    mask = (gids == batch_ref[...]).astype(jnp.float32)
    sums = jnp.dot(mask, h, preferred_element_type=jnp.float32)
    cnt = jnp.sum(mask, axis=1, keepdims=True)
    pooled = sums / jnp.maximum(cnt, 1.0)
    z = _elu(jnp.dot(pooled, fw1_ref[...], preferred_element_type=jnp.float32)
             + fb1_ref[...])
    z = _elu(jnp.dot(z, fw2_ref[...], preferred_element_type=jnp.float32)
             + fb2_ref[...])
    o_ref[...] = jnp.dot(z, fw3_ref[...],
                         preferred_element_type=jnp.float32) + fb3_ref[...]


def kernel(x, edge_index, batch, pre_W1, pre_b1, bn_g, bn_b, pre_W2, pre_b2,
           mid_W1, mid_b1, mid_W2, mid_b2, post_W1, post_b1, post_W2, post_b2,
           f_W1, f_b1, f_W2, f_b2, f_W3, f_b3):
    pad = NC_T * CH - EPT
    src2 = jnp.pad(edge_index[0].reshape(NTILES, EPT),
                   ((0, 0), (0, pad))).reshape(NTILES, NC_T, CH)
    dst2 = jnp.pad(edge_index[1].reshape(NTILES, EPT),
                   ((0, 0), (0, pad)),
                   constant_values=N).reshape(NTILES, NC_T, CH)
    zrows = jnp.zeros((ROWS_PER_TILE, H), jnp.bfloat16)
    batch_row = batch.reshape(1, N)
    r1 = lambda v: v.reshape(1, -1)

    y, yb = _tc(_proj_body, _dual((N, H)), x, pre_W1)
    p = _sc_segsum(yb, src2, dst2, zrows)
    h1, h1b = _tc(_pre_body, _dual((N, H)), y, p, r1(pre_b1), r1(bn_g),
                  r1(bn_b), pre_W2, r1(pre_b2))
    q = _sc_segsum(h1b, src2, dst2, zrows)
    h2, h2b = _tc(_mid_body, _dual((N, H)), h1, q, mid_W1, r1(mid_b1),
                  mid_W2, r1(mid_b2))
    r = _sc_segsum(h2b, src2, dst2, zrows)
    out = _tc(_post_body, _f32((G, OUT)), h2, r, batch_row,
              post_W1, r1(post_b1), post_W2, r1(post_b2),
              f_W1, r1(f_b1), f_W2, r1(f_b2), f_W3, r1(f_b3))
    return out
